# baseline (speedup 1.0000x reference)
import functools
import math

import jax
import jax.numpy as jnp
from jax import lax
from jax.experimental import pallas as pl
from jax.experimental.pallas import tpu as pltpu
from jax.experimental.pallas import tpu_sc as plsc

VOCAB = 1000000
D = 32
L_SEQ = 200
B_ROWS = 16384
N_FLAT = B_ROWS * L_SEQ
NUM_CORES = 2
NUM_SUBCORES = 16
NW = NUM_CORES * NUM_SUBCORES
PER_W = N_FLAT // NW
NBUF = 4
CHUNK = 512
STEPS = PER_W // CHUNK
ITERS = STEPS // NBUF
SCALE = math.sqrt(float(D))

_mesh = plsc.VectorSubcoreMesh(
    core_axis_name="c", subcore_axis_name="s",
    num_cores=NUM_CORES, num_subcores=NUM_SUBCORES)


@functools.partial(
    pl.kernel,
    out_type=jax.ShapeDtypeStruct((L_SEQ, D, B_ROWS), jnp.float32),
    mesh=_mesh,
    scratch_types=[
        [pltpu.VMEM((CHUNK,), jnp.int32) for _ in range(NBUF)],
        [pltpu.VMEM((CHUNK, D), jnp.float32) for _ in range(NBUF)],
        [pltpu.VMEM((D, CHUNK), jnp.float32) for _ in range(2)],
        pltpu.VMEM((8, D, 16), jnp.float32),
        [pltpu.SemaphoreType.DMA for _ in range(NBUF)],
        [pltpu.SemaphoreType.DMA for _ in range(2)],
    ],
    compiler_params=pltpu.CompilerParams(use_tc_tiling_on_sc=False,
                                         needs_layout_passes=False),
)
def _emb_lookup(x_hbm, table_hbm, pos_splat_hbm, out_hbm,
                idx_v, rows_v, ob, psb, sg, so):
    wid = lax.axis_index("s") * NUM_CORES + lax.axis_index("c")
    base = wid * PER_W
    l_first = base // B_ROWS
    pltpu.sync_copy(pos_splat_hbm.at[pl.ds(l_first, 8)], psb)
    iota16 = jnp.arange(16, dtype=jnp.int32)

    def out_slice(g):
        off = base + g * CHUNK
        return out_hbm.at[off // B_ROWS, :, pl.ds(off % B_ROWS, CHUNK)]

    def transpose_fma(rows, o, l):
        dl = l - l_first

        def dloop(d, c):
            psv = psb[dl, d, pl.ds(0, 16)]
            dsplat = jnp.full((16,), d, dtype=jnp.int32)

            def jloop(j, c2):
                for u in range(4):
                    bi = iota16 + (j * 4 + u) * 16
                    col = plsc.load_gather(rows, [bi, dsplat])
                    o[d, pl.ds((j * 4 + u) * 16, 16)] = col * SCALE + psv
                return c2
            lax.fori_loop(0, CHUNK // 64, jloop, 0)
            return c
        lax.fori_loop(0, D, dloop, 0)

    for b in range(2):
        pltpu.sync_copy(x_hbm.at[pl.ds(base + b * CHUNK, CHUNK)], idx_v[b])
        pltpu.async_copy(table_hbm.at[idx_v[b]], rows_v[b], sg[b])

    def ring(i, carry):
        for s in range(NBUF):
            g = i * NBUF + s
            off = base + g * CHUNK
            ot = s % 2
            pltpu.make_async_copy(table_hbm.at[idx_v[s]], rows_v[s],
                                  sg[s]).wait()

            @pl.when(g >= 2)
            def _():
                pltpu.make_async_copy(ob[ot], out_slice(g - 2), so[ot]).wait()

            transpose_fma(rows_v[s], ob[ot], off // B_ROWS)
            pltpu.async_copy(ob[ot], out_slice(g), so[ot])

            t = (s + 2) % NBUF
            gp = g + 2

            @pl.when(gp < STEPS)
            def _():
                pltpu.sync_copy(
                    x_hbm.at[pl.ds(base + gp * CHUNK, CHUNK)], idx_v[t])
                pltpu.async_copy(table_hbm.at[idx_v[t]], rows_v[t], sg[t])
        return carry

    lax.fori_loop(0, ITERS, ring, 0)

    for g in (STEPS - 2, STEPS - 1):
        pltpu.make_async_copy(ob[g % 2], out_slice(g), so[g % 2]).wait()


def kernel(x, table, pos_encoding):
    x_lmaj = x.T.reshape(-1).astype(jnp.int32)
    pos_splat = jnp.broadcast_to(pos_encoding[:208, :, None], (208, D, 16))
    out = _emb_lookup(x_lmaj, table, pos_splat)
    return out.transpose(2, 0, 1)
